# baseline (speedup 1.0000x reference)
import functools

import jax
import jax.numpy as jnp
from jax import lax
from jax.experimental import pallas as pl
from jax.experimental.pallas import tpu as pltpu
from jax.experimental.pallas import tpu_sc as plsc

B, H, W = 16, 512, 512
NC, NS = 2, 16
LANES = 16
CH = 64
HALF = H // 2
NCHUNK = HALF // CH
NCG = W // LANES


def _att_map_sc(mask, thr_vec, meta):
    mesh = plsc.VectorSubcoreMesh(core_axis_name="c", subcore_axis_name="s")

    @functools.partial(
        pl.kernel,
        mesh=mesh,
        out_type=jax.ShapeDtypeStruct((B, H, W), jnp.float32),
        scratch_types=[
            pltpu.VMEM((CH, W), jnp.float32),
            pltpu.VMEM((CH, W), jnp.float32),
            pltpu.VMEM((W,), jnp.float32),
            pltpu.VMEM((LANES,), jnp.float32),
            pltpu.VMEM((2, LANES), jnp.int32),
            pltpu.VMEM((5, LANES), jnp.int32),
            pltpu.VMEM((5, LANES), jnp.int32),
            pltpu.VMEM_SHARED((NS, 5, LANES), jnp.int32),
            pltpu.SemaphoreType.DMA,
            pltpu.SemaphoreType.DMA,
            pltpu.SemaphoreType.DMA,
            pltpu.SemaphoreType.DMA,
        ],
    )
    def sc_kernel(mask_hbm, thr_hbm, meta_hbm, out_hbm,
                  buf0, buf1, prow, thr_v, meta_v, pacc, pacc2, shared,
                  isem0, isem1, osem0, osem1):
        c = lax.axis_index("c")
        s = lax.axis_index("s")
        b = c * (B // NC) + (s >> 1)
        row0 = (s & 1) * HALF

        pltpu.sync_copy(thr_hbm, thr_v)
        pltpu.sync_copy(meta_hbm, meta_v)
        thr = thr_v[...]
        npts_v = meta_v[0, :]
        dist_v = meta_v[1, :]

        xio = lax.iota(jnp.int32, LANES)
        bufs = (buf0, buf1)
        isems = (isem0, isem1)
        osems = (osem0, osem1)

        handles = [None, None]
        handles[0] = pltpu.async_copy(
            mask_hbm.at[b, pl.ds(row0, CH)], buf0, isem0)
        carry = (
            jnp.zeros((LANES,), jnp.int32),
            jnp.full((LANES,), W, jnp.int32),
            jnp.full((LANES,), -1, jnp.int32),
            jnp.full((LANES,), H, jnp.int32),
            jnp.full((LANES,), -1, jnp.int32),
        )
        for kc in range(NCHUNK):
            nxt = kc + 1
            if nxt < NCHUNK:
                handles[nxt & 1] = pltpu.async_copy(
                    mask_hbm.at[b, pl.ds(row0 + nxt * CH, CH)],
                    bufs[nxt & 1], isems[nxt & 1])
            handles[kc & 1].wait()
            buf = bufs[kc & 1]
            ybase = row0 + kc * CH

            def row_body(r, acc, buf=buf, ybase=ybase):
                cnt, mnx, mxx, mny, mxy = acc
                y = ybase + r
                rowor = None
                for cc in range(NCG):
                    v = buf[r, pl.ds(cc * LANES, LANES)]
                    m = v > thr
                    x = cc * LANES + xio
                    cnt = cnt + jnp.where(m, jnp.int32(1), jnp.int32(0))
                    mnx = jnp.minimum(mnx, jnp.where(m, x, jnp.int32(W)))
                    mxx = jnp.maximum(mxx, jnp.where(m, x, jnp.int32(-1)))
                    rowor = m if rowor is None else (rowor | m)
                mny = jnp.minimum(mny, jnp.where(rowor, y, jnp.int32(H)))
                mxy = jnp.maximum(mxy, jnp.where(rowor, y, jnp.int32(-1)))
                return (cnt, mnx, mxx, mny, mxy)

            carry = lax.fori_loop(0, CH, row_body, carry)

        cnt, mnx, mxx, mny, mxy = carry
        pacc[0, :] = cnt
        pacc[1, :] = mnx
        pacc[2, :] = mxx
        pacc[3, :] = mny
        pacc[4, :] = mxy

        pltpu.sync_copy(pacc, shared.at[s])
        plsc.subcore_barrier()
        pltpu.sync_copy(shared.at[s ^ 1], pacc2)

        cnt_m = cnt + pacc2[0, :]
        mnx_m = jnp.minimum(mnx, pacc2[1, :])
        mxx_m = jnp.maximum(mxx, pacc2[2, :])
        mny_m = jnp.minimum(mny, pacc2[3, :])
        mxy_m = jnp.maximum(mxy, pacc2[4, :])

        def shuffle(x, perm):
            return lax.gather(
                x, perm[:, None],
                lax.GatherDimensionNumbers(
                    offset_dims=(), collapsed_slice_dims=(0,),
                    start_index_map=(0,)),
                (1,), mode=lax.GatherScatterMode.PROMISE_IN_BOUNDS)

        for sh in (1, 2, 4, 8):
            perm = (xio + sh) & (LANES - 1)
            cnt_m = cnt_m + shuffle(cnt_m, perm)
            mnx_m = jnp.minimum(mnx_m, shuffle(mnx_m, perm))
            mxx_m = jnp.maximum(mxx_m, shuffle(mxx_m, perm))
            mny_m = jnp.minimum(mny_m, shuffle(mny_m, perm))
            mxy_m = jnp.maximum(mxy_m, shuffle(mxy_m, perm))

        min_y = jnp.clip(mny_m - dist_v, 0, H - 1)
        max_y = jnp.clip(mxy_m + dist_v, 0, H - 1)
        min_x = jnp.clip(mnx_m - dist_v, 0, W - 1)
        max_x = jnp.clip(mxx_m + dist_v, 0, W - 1)
        fallback = cnt_m < npts_v

        def prow_body(cc, t):
            x = cc * LANES + xio
            px = (x >= min_x) & (x <= max_x)
            prow[pl.ds(cc * LANES, LANES)] = jnp.where(
                fallback | px, jnp.float32(1.0), jnp.float32(0.0))
            return t
        lax.fori_loop(0, NCG, prow_body, 0)
        bgv = jnp.where(fallback, jnp.float32(1.0), jnp.float32(0.0))

        oh = [None, None]
        for kc in range(NCHUNK):
            if kc >= 2:
                oh[kc & 1].wait()
            buf = bufs[kc & 1]
            ybase = row0 + kc * CH

            def row2(r, t, buf=buf, ybase=ybase):
                yv = jnp.full((LANES,), ybase + r, jnp.int32)
                iny = (yv >= min_y) & (yv <= max_y)
                for cc in range(NCG):
                    pv = prow[pl.ds(cc * LANES, LANES)]
                    buf[r, pl.ds(cc * LANES, LANES)] = jnp.where(iny, pv, bgv)
                return t

            lax.fori_loop(0, CH, row2, 0)
            oh[kc & 1] = pltpu.async_copy(
                buf, out_hbm.at[b, pl.ds(ybase, CH)], osems[kc & 1])
        oh[0].wait()
        oh[1].wait()

    return sc_kernel(mask, thr_vec, meta)


def kernel(mask, prob_threshold=0.5, n_pts_threshold=10, dist_threshold=64):
    thr_vec = jnp.full((LANES,), prob_threshold, jnp.float32)
    meta = jnp.stack([
        jnp.full((LANES,), n_pts_threshold, jnp.int32),
        jnp.full((LANES,), dist_threshold, jnp.int32),
    ])
    return _att_map_sc(mask.astype(jnp.float32), thr_vec, meta)


# speedup vs baseline: 1.6891x; 1.6891x over previous
import functools

import jax
import jax.numpy as jnp
from jax import lax
from jax.experimental import pallas as pl
from jax.experimental.pallas import tpu as pltpu
from jax.experimental.pallas import tpu_sc as plsc

B, H, W = 16, 512, 512
NC, NS = 2, 16
LANES = 16
CH = 64
HALF = H // 2
NCHUNK = HALF // CH
NCG = W // LANES


def _att_map_sc(mask, thr_vec, meta):
    mesh = plsc.VectorSubcoreMesh(core_axis_name="c", subcore_axis_name="s")

    @functools.partial(
        pl.kernel,
        mesh=mesh,
        out_type=jax.ShapeDtypeStruct((B, H, W), jnp.float32),
        scratch_types=[
            pltpu.VMEM((CH, W), jnp.float32),
            pltpu.VMEM((CH, W), jnp.float32),
            pltpu.VMEM((W,), jnp.float32),
            pltpu.VMEM((LANES,), jnp.float32),
            pltpu.VMEM((2, LANES), jnp.int32),
            pltpu.VMEM((5, LANES), jnp.int32),
            pltpu.VMEM((5, LANES), jnp.int32),
            pltpu.VMEM_SHARED((NS, 5, LANES), jnp.int32),
            pltpu.SemaphoreType.DMA,
            pltpu.SemaphoreType.DMA,
            pltpu.SemaphoreType.DMA,
            pltpu.SemaphoreType.DMA,
        ],
    )
    def sc_kernel(mask_hbm, thr_hbm, meta_hbm, out_hbm,
                  buf0, buf1, prow, thr_v, meta_v, pacc, pacc2, shared,
                  isem0, isem1, osem0, osem1):
        c = lax.axis_index("c")
        s = lax.axis_index("s")
        b = c * (B // NC) + (s >> 1)
        row0 = (s & 1) * HALF

        pltpu.sync_copy(thr_hbm, thr_v)
        pltpu.sync_copy(meta_hbm, meta_v)
        thr = thr_v[...]
        npts_v = meta_v[0, :]
        dist_v = meta_v[1, :]

        xio = lax.iota(jnp.int32, LANES)
        bufs = (buf0, buf1)
        isems = (isem0, isem1)
        osems = (osem0, osem1)

        handles = [None, None]
        handles[0] = pltpu.async_copy(
            mask_hbm.at[b, pl.ds(row0, CH)], buf0, isem0)
        carry = (
            jnp.zeros((LANES,), jnp.int32),
            jnp.full((LANES,), H, jnp.int32),
            jnp.full((LANES,), -1, jnp.int32),
        ) + tuple(
            jnp.full((LANES,), -jnp.inf, jnp.float32) for _ in range(NCG)
        )
        for kc in range(NCHUNK):
            nxt = kc + 1
            if nxt < NCHUNK:
                handles[nxt & 1] = pltpu.async_copy(
                    mask_hbm.at[b, pl.ds(row0 + nxt * CH, CH)],
                    bufs[nxt & 1], isems[nxt & 1])
            handles[kc & 1].wait()
            buf = bufs[kc & 1]
            ybase = row0 + kc * CH

            def row_body(r, acc, buf=buf, ybase=ybase):
                cnt, mny, mxy = acc[0], acc[1], acc[2]
                cms = list(acc[3:])
                y = ybase + r
                rc = jnp.zeros((LANES,), jnp.int32)
                for cc in range(NCG):
                    v = buf[r, pl.ds(cc * LANES, LANES)]
                    m = v > thr
                    rc = rc + jnp.where(m, jnp.int32(1), jnp.int32(0))
                    cms[cc] = jnp.maximum(cms[cc], v)
                pos = rc > jnp.int32(0)
                cnt = cnt + rc
                mny = jnp.minimum(mny, jnp.where(pos, y, jnp.int32(H)))
                mxy = jnp.maximum(mxy, jnp.where(pos, y, jnp.int32(-1)))
                return (cnt, mny, mxy, *cms)

            carry = lax.fori_loop(0, CH, row_body, carry)

        cnt, mny, mxy = carry[0], carry[1], carry[2]
        mnx = jnp.full((LANES,), W, jnp.int32)
        mxx = jnp.full((LANES,), -1, jnp.int32)
        for cc in range(NCG):
            mc = carry[3 + cc] > thr
            x = cc * LANES + xio
            mnx = jnp.minimum(mnx, jnp.where(mc, x, jnp.int32(W)))
            mxx = jnp.maximum(mxx, jnp.where(mc, x, jnp.int32(-1)))
        pacc[0, :] = cnt
        pacc[1, :] = mnx
        pacc[2, :] = mxx
        pacc[3, :] = mny
        pacc[4, :] = mxy

        pltpu.sync_copy(pacc, shared.at[s])
        plsc.subcore_barrier()
        pltpu.sync_copy(shared.at[s ^ 1], pacc2)

        cnt_m = cnt + pacc2[0, :]
        mnx_m = jnp.minimum(mnx, pacc2[1, :])
        mxx_m = jnp.maximum(mxx, pacc2[2, :])
        mny_m = jnp.minimum(mny, pacc2[3, :])
        mxy_m = jnp.maximum(mxy, pacc2[4, :])

        def shuffle(x, perm):
            return lax.gather(
                x, perm[:, None],
                lax.GatherDimensionNumbers(
                    offset_dims=(), collapsed_slice_dims=(0,),
                    start_index_map=(0,)),
                (1,), mode=lax.GatherScatterMode.PROMISE_IN_BOUNDS)

        for sh in (1, 2, 4, 8):
            perm = (xio + sh) & (LANES - 1)
            cnt_m = cnt_m + shuffle(cnt_m, perm)
            mnx_m = jnp.minimum(mnx_m, shuffle(mnx_m, perm))
            mxx_m = jnp.maximum(mxx_m, shuffle(mxx_m, perm))
            mny_m = jnp.minimum(mny_m, shuffle(mny_m, perm))
            mxy_m = jnp.maximum(mxy_m, shuffle(mxy_m, perm))

        min_y = jnp.clip(mny_m - dist_v, 0, H - 1)
        max_y = jnp.clip(mxy_m + dist_v, 0, H - 1)
        min_x = jnp.clip(mnx_m - dist_v, 0, W - 1)
        max_x = jnp.clip(mxx_m + dist_v, 0, W - 1)
        fallback = cnt_m < npts_v

        def prow_body(cc, t):
            x = cc * LANES + xio
            px = (x >= min_x) & (x <= max_x)
            prow[pl.ds(cc * LANES, LANES)] = jnp.where(
                fallback | px, jnp.float32(1.0), jnp.float32(0.0))
            return t
        lax.fori_loop(0, NCG, prow_body, 0)
        bgv = jnp.where(fallback, jnp.float32(1.0), jnp.float32(0.0))

        oh = [None, None]
        for kc in range(NCHUNK):
            if kc >= 2:
                oh[kc & 1].wait()
            buf = bufs[kc & 1]
            ybase = row0 + kc * CH

            def row2(r, t, buf=buf, ybase=ybase):
                yv = jnp.full((LANES,), ybase + r, jnp.int32)
                iny = (yv >= min_y) & (yv <= max_y)
                for cc in range(NCG):
                    pv = prow[pl.ds(cc * LANES, LANES)]
                    buf[r, pl.ds(cc * LANES, LANES)] = jnp.where(iny, pv, bgv)
                return t

            lax.fori_loop(0, CH, row2, 0)
            oh[kc & 1] = pltpu.async_copy(
                buf, out_hbm.at[b, pl.ds(ybase, CH)], osems[kc & 1])
        oh[0].wait()
        oh[1].wait()

    return sc_kernel(mask, thr_vec, meta)


def kernel(mask, prob_threshold=0.5, n_pts_threshold=10, dist_threshold=64):
    thr_vec = jnp.full((LANES,), prob_threshold, jnp.float32)
    meta = jnp.stack([
        jnp.full((LANES,), n_pts_threshold, jnp.int32),
        jnp.full((LANES,), dist_threshold, jnp.int32),
    ])
    return _att_map_sc(mask.astype(jnp.float32), thr_vec, meta)
